# baseline (speedup 1.0000x reference)
import functools

import jax
import jax.numpy as jnp
from jax import lax
from jax.experimental import pallas as pl
from jax.experimental.pallas import tpu as pltpu
from jax.experimental.pallas import tpu_sc as plsc

T = 2048
D = 1024
F = 4096
E = 8
K = 2
A = T * K
B_TILE = 256
P = 6144
NT = P // B_TILE
NT_PAD = 32
NC = 4
FC = F // NC
NW = 32
ROWS_W = P // NW
GCH = 32
TOK_W = T // NW
TCH = 32


def _cumsum0(m):
    n = m.shape[0]
    s = 1
    while s < n:
        shifted = jnp.concatenate(
            [jnp.zeros((s, m.shape[1]), m.dtype), m[: n - s, :]], axis=0)
        m = m + shifted
        s *= 2
    return m


def _plan_body(lg_ref, posf_ref, wf_ref, te_ref):
    logits = lg_ref[...]
    lane = lax.broadcasted_iota(jnp.int32, (T, E), 1)
    m0 = jnp.max(logits, axis=1, keepdims=True)
    i0 = jnp.min(jnp.where(logits == m0, lane, E), axis=1, keepdims=True)
    masked = jnp.where(lane == i0, jnp.float32(-1e30), logits)
    m1 = jnp.max(masked, axis=1, keepdims=True)
    i1 = jnp.min(jnp.where(masked == m1, lane, E), axis=1, keepdims=True)
    w0 = 1.0 / (1.0 + jnp.exp(m1 - m0))
    wf_ref[...] = jnp.concatenate([w0, 1.0 - w0], axis=0)

    oh0 = (i0 == lane).astype(jnp.int32)
    oh1 = (i1 == lane).astype(jnp.int32)
    c0 = _cumsum0(oh0)
    cnt0 = c0[T - 1:T, :]
    c1 = _cumsum0(oh1) + cnt0
    cnt = c1[T - 1:T, :]
    tiles = (cnt + (B_TILE - 1)) // B_TILE

    sub8 = lax.broadcasted_iota(jnp.int32, (E, E), 0)
    lan8 = lax.broadcasted_iota(jnp.int32, (E, E), 1)
    tiles_col = jnp.sum(
        jnp.where(sub8 == lan8, jnp.broadcast_to(tiles, (E, E)), 0),
        axis=1, keepdims=True)
    tile_start = jnp.sum(
        jnp.where(sub8 < lan8, jnp.broadcast_to(tiles_col, (E, E)), 0),
        axis=0, keepdims=True)
    pstart = tile_start * B_TILE
    n_act = jnp.sum(tiles, axis=1, keepdims=True)

    rank0 = jnp.sum(oh0 * (c0 - 1), axis=1, keepdims=True)
    base0 = jnp.sum(oh0 * jnp.broadcast_to(pstart, (T, E)), axis=1,
                    keepdims=True)
    rank1 = jnp.sum(oh1 * (c1 - 1), axis=1, keepdims=True)
    base1 = jnp.sum(oh1 * jnp.broadcast_to(pstart, (T, E)), axis=1,
                    keepdims=True)
    posf_ref[...] = jnp.concatenate([rank0 + base0, rank1 + base1], axis=0)

    t_iota = lax.broadcasted_iota(jnp.int32, (NT_PAD, E), 0)
    te = jnp.sum((t_iota >= jnp.broadcast_to(tile_start, (NT_PAD, E)))
                 .astype(jnp.int32), axis=1, keepdims=True) - 1
    te = jnp.clip(te, 0, E - 1)
    r_iota = lax.broadcasted_iota(jnp.int32, (NT_PAD, 1), 0)
    te_full = jnp.where(r_iota == NT, jnp.broadcast_to(n_act, (NT_PAD, 1)),
                        jnp.where(r_iota < NT, te, 0))
    te_ref[...] = te_full


def _plan_call(logits, interpret=False):
    out_shape = (
        jax.ShapeDtypeStruct((A, 1), jnp.int32),
        jax.ShapeDtypeStruct((A, 1), jnp.float32),
        jax.ShapeDtypeStruct((NT_PAD, 1), jnp.int32),
    )
    return pl.pallas_call(_plan_body, out_shape=out_shape,
                          interpret=interpret)(logits)


def _dispatch_body(pos_hbm, x_hbm, te_hbm, xs_hbm,
                   pos_v, tok_v, nt_v, buf0, buf1, sem0, sem1):
    wid = lax.axis_index("s") * 2 + lax.axis_index("c")
    base = wid * ROWS_W
    pltpu.sync_copy(pos_hbm, pos_v)
    pltpu.sync_copy(te_hbm, nt_v)
    n_total = nt_v[pl.ds(16, 16)][NT - 16] * B_TILE

    zi = jnp.zeros((16,), jnp.int32)

    def _zero(i, _):
        tok_v[pl.ds(base + i * 16, 16)] = zi
        return 0
    lax.fori_loop(0, ROWS_W // 16, _zero, 0, unroll=4)

    lane16 = lax.iota(jnp.int32, 16)

    def _scatter(i, _):
        idx = pos_v[pl.ds(i * 16, 16)]
        a = lane16 + i * 16
        tok = lax.bitwise_and(a, T - 1)
        plsc.store_scatter(tok_v, [idx], tok)
        return 0
    lax.fori_loop(0, A // 16, _scatter, 0, unroll=4)

    active = n_total - base
    bufs = (buf0, buf1)
    sems = (sem0, sem1)
    nch = ROWS_W // GCH

    def _src(i):
        return x_hbm.at[tok_v.at[pl.ds(base + i * GCH, GCH)]]

    def _drain_out(j):
        @pl.when(j * GCH < active)
        def _():
            pltpu.make_async_copy(_src(j), bufs[j % 2], sems[j % 2]).wait()
            pltpu.sync_copy(bufs[j % 2],
                            xs_hbm.at[pl.ds(base + j * GCH, GCH)])

    for i in range(nch):
        @pl.when(i * GCH < active)
        def _(i=i):
            pltpu.async_copy(_src(i), bufs[i % 2], sems[i % 2])
        if i >= 1:
            _drain_out(i - 1)
    _drain_out(nch - 1)


def _dispatch_call(pos_flat, flat, te_flat):
    mesh = plsc.VectorSubcoreMesh(core_axis_name="c", subcore_axis_name="s")
    f = pl.kernel(
        _dispatch_body,
        out_type=jax.ShapeDtypeStruct((P, D), jnp.float32),
        mesh=mesh,
        scratch_types=[
            pltpu.VMEM((A,), jnp.int32),
            pltpu.VMEM((P,), jnp.int32),
            pltpu.VMEM((NT_PAD,), jnp.int32),
            pltpu.VMEM((GCH, D), jnp.float32),
            pltpu.VMEM((GCH, D), jnp.float32),
            pltpu.SemaphoreType.DMA,
            pltpu.SemaphoreType.DMA,
        ],
        compiler_params=pltpu.CompilerParams(needs_layout_passes=False))
    return f(pos_flat, flat, te_flat)


def _mlp_body(te_s, na_s, xs_ref, wg_ref, wi_ref, wo_ref,
              out_ref, acc_ref):
    c = pl.program_id(0)
    t = pl.program_id(1)

    @pl.when(t < na_s[0])
    def _():
        xb = xs_ref[...].astype(jnp.bfloat16)
        g = jnp.dot(xb, wg_ref[0].astype(jnp.bfloat16),
                    preferred_element_type=jnp.float32)
        u = jnp.dot(xb, wi_ref[0].astype(jnp.bfloat16),
                    preferred_element_type=jnp.float32)
        h = (g * jax.lax.logistic(g)) * u
        part = jnp.dot(h.astype(jnp.bfloat16), wo_ref[0].astype(jnp.bfloat16),
                       preferred_element_type=jnp.float32)
        sl = pl.ds(t * B_TILE, B_TILE)

        @pl.when(c == 0)
        def _a():
            acc_ref[sl, :] = part

        @pl.when(c > 0)
        def _b():
            acc_ref[sl, :] = acc_ref[sl, :] + part

        @pl.when(c == NC - 1)
        def _c():
            out_ref[...] = acc_ref[sl, :]


def _mlp_call(te_arr, na_arr, xs, wg, wi, wo, interpret=False):
    grid_spec = pltpu.PrefetchScalarGridSpec(
        num_scalar_prefetch=2,
        grid=(NC, NT),
        in_specs=[
            pl.BlockSpec((B_TILE, D), lambda c, t, te, na: (t, 0)),
            pl.BlockSpec((1, D, FC), lambda c, t, te, na: (te[t], 0, c)),
            pl.BlockSpec((1, D, FC), lambda c, t, te, na: (te[t], 0, c)),
            pl.BlockSpec((1, FC, D), lambda c, t, te, na: (te[t], c, 0)),
        ],
        out_specs=pl.BlockSpec((B_TILE, D), lambda c, t, te, na: (t, 0)),
        scratch_shapes=[pltpu.VMEM((P, D), jnp.float32)],
    )
    return pl.pallas_call(
        _mlp_body,
        grid_spec=grid_spec,
        out_shape=jax.ShapeDtypeStruct((P, D), jnp.float32),
        compiler_params=pltpu.CompilerParams(
            dimension_semantics=("arbitrary", "arbitrary"),
            vmem_limit_bytes=100 * 1024 * 1024),
        interpret=interpret,
    )(te_arr, na_arr, xs, wg, wi, wo)


def _combine_body(posf_hbm, wf_hbm, ys_hbm, out_hbm,
                  p0_v, p1_v, w0_v, w1_v, r0, r1, sem0, sem1):
    wid = lax.axis_index("s") * 2 + lax.axis_index("c")
    base = wid * TOK_W

    def chunk(cix, _):
        off = base + cix * TCH
        pltpu.sync_copy(posf_hbm.at[pl.ds(off, TCH)], p0_v)
        pltpu.sync_copy(posf_hbm.at[pl.ds(T + off, TCH)], p1_v)
        pltpu.sync_copy(wf_hbm.at[pl.ds(off, TCH)], w0_v)
        pltpu.sync_copy(wf_hbm.at[pl.ds(T + off, TCH)], w1_v)
        cp0 = pltpu.async_copy(ys_hbm.at[p0_v], r0, sem0)
        cp1 = pltpu.async_copy(ys_hbm.at[p1_v], r1, sem1)
        cp0.wait()
        cp1.wait()

        for g in range(TCH // 16):
            w0vec = w0_v[pl.ds(g * 16, 16)]
            w1vec = w1_v[pl.ds(g * 16, 16)]
            for j2 in range(16):
                j = g * 16 + j2
                s0 = w0vec[j2]
                s1 = w1vec[j2]

                def vec(v, _, j=j, s0=s0, s1=s1):
                    sl = pl.ds(v * 16, 16)
                    r0[j, sl] = s0 * r0[j, sl] + s1 * r1[j, sl]
                    return 0
                lax.fori_loop(0, D // 16, vec, 0, unroll=4)
        pltpu.sync_copy(r0, out_hbm.at[pl.ds(off, TCH)])
        return 0
    lax.fori_loop(0, TOK_W // TCH, chunk, 0)


def _combine_call(pos_flat, w_flat, ys):
    mesh = plsc.VectorSubcoreMesh(core_axis_name="c", subcore_axis_name="s")
    f = pl.kernel(
        _combine_body,
        out_type=jax.ShapeDtypeStruct((T, D), jnp.float32),
        mesh=mesh,
        scratch_types=[
            pltpu.VMEM((TCH,), jnp.int32),
            pltpu.VMEM((TCH,), jnp.int32),
            pltpu.VMEM((TCH,), jnp.float32),
            pltpu.VMEM((TCH,), jnp.float32),
            pltpu.VMEM((TCH, D), jnp.float32),
            pltpu.VMEM((TCH, D), jnp.float32),
            pltpu.SemaphoreType.DMA,
            pltpu.SemaphoreType.DMA,
        ],
        compiler_params=pltpu.CompilerParams(needs_layout_passes=False))
    return f(pos_flat, w_flat, ys)


def kernel(x, W_router, W_gate, W_in, W_out):
    Bb, S, Dm = x.shape
    flat = x.reshape(T, D)
    logits = flat @ W_router
    posf, wf, te_full = _plan_call(logits)
    pos_flat = posf[:, 0]
    w_flat = wf[:, 0]
    xs = _dispatch_call(pos_flat, flat, te_full[:, 0])
    te_arr = te_full[:NT, 0]
    na_arr = te_full[NT:NT + 1, 0]
    ys = _mlp_call(te_arr, na_arr, xs, W_gate, W_in, W_out)
    out = _combine_call(pos_flat, w_flat, ys)
    return out.reshape(Bb, S, Dm)


# speedup vs baseline: 1.0132x; 1.0132x over previous
import functools

import jax
import jax.numpy as jnp
from jax import lax
from jax.experimental import pallas as pl
from jax.experimental.pallas import tpu as pltpu
from jax.experimental.pallas import tpu_sc as plsc

T = 2048
D = 1024
F = 4096
E = 8
K = 2
A = T * K
B_TILE = 256
P = 6144
NT = P // B_TILE
NT_PAD = 32
NC = 4
FC = F // NC
NW = 32
ROWS_W = P // NW
GCH = 32
TOK_W = T // NW
TCH = 32


def _cumsum0(m):
    n = m.shape[0]
    s = 1
    while s < n:
        shifted = jnp.concatenate(
            [jnp.zeros((s, m.shape[1]), m.dtype), m[: n - s, :]], axis=0)
        m = m + shifted
        s *= 2
    return m


def _plan_body(lg_ref, posf_ref, wf_ref, te_ref):
    logits = lg_ref[...]
    lane = lax.broadcasted_iota(jnp.int32, (T, E), 1)
    m0 = jnp.max(logits, axis=1, keepdims=True)
    i0 = jnp.min(jnp.where(logits == m0, lane, E), axis=1, keepdims=True)
    masked = jnp.where(lane == i0, jnp.float32(-1e30), logits)
    m1 = jnp.max(masked, axis=1, keepdims=True)
    i1 = jnp.min(jnp.where(masked == m1, lane, E), axis=1, keepdims=True)
    w0 = 1.0 / (1.0 + jnp.exp(m1 - m0))
    wf_ref[...] = jnp.concatenate([w0, 1.0 - w0], axis=0)

    oh0 = (i0 == lane).astype(jnp.int32)
    oh1 = (i1 == lane).astype(jnp.int32)
    c0 = _cumsum0(oh0)
    cnt0 = c0[T - 1:T, :]
    c1 = _cumsum0(oh1) + cnt0
    cnt = c1[T - 1:T, :]
    tiles = (cnt + (B_TILE - 1)) // B_TILE

    sub8 = lax.broadcasted_iota(jnp.int32, (E, E), 0)
    lan8 = lax.broadcasted_iota(jnp.int32, (E, E), 1)
    tiles_col = jnp.sum(
        jnp.where(sub8 == lan8, jnp.broadcast_to(tiles, (E, E)), 0),
        axis=1, keepdims=True)
    tile_start = jnp.sum(
        jnp.where(sub8 < lan8, jnp.broadcast_to(tiles_col, (E, E)), 0),
        axis=0, keepdims=True)
    pstart = tile_start * B_TILE
    n_act = jnp.sum(tiles, axis=1, keepdims=True)

    rank0 = jnp.sum(oh0 * (c0 - 1), axis=1, keepdims=True)
    base0 = jnp.sum(oh0 * jnp.broadcast_to(pstart, (T, E)), axis=1,
                    keepdims=True)
    rank1 = jnp.sum(oh1 * (c1 - 1), axis=1, keepdims=True)
    base1 = jnp.sum(oh1 * jnp.broadcast_to(pstart, (T, E)), axis=1,
                    keepdims=True)
    posf_ref[...] = jnp.concatenate([rank0 + base0, rank1 + base1], axis=0)

    t_iota = lax.broadcasted_iota(jnp.int32, (NT_PAD, E), 0)
    te = jnp.sum((t_iota >= jnp.broadcast_to(tile_start, (NT_PAD, E)))
                 .astype(jnp.int32), axis=1, keepdims=True) - 1
    te = jnp.clip(te, 0, E - 1)
    r_iota = lax.broadcasted_iota(jnp.int32, (NT_PAD, 1), 0)
    te_full = jnp.where(r_iota == NT, jnp.broadcast_to(n_act, (NT_PAD, 1)),
                        jnp.where(r_iota < NT, te, 0))
    te_ref[...] = te_full


def _plan_call(logits, interpret=False):
    out_shape = (
        jax.ShapeDtypeStruct((A, 1), jnp.int32),
        jax.ShapeDtypeStruct((A, 1), jnp.float32),
        jax.ShapeDtypeStruct((NT_PAD, 1), jnp.int32),
    )
    return pl.pallas_call(_plan_body, out_shape=out_shape,
                          interpret=interpret)(logits)


def _dispatch_body(pos_hbm, w_hbm, x_hbm, te_hbm, xs_hbm, wrow_hbm,
                   pos_v, w_v, tok_v, wr_v, nt_v, buf0, buf1, sem0, sem1):
    wid = lax.axis_index("s") * 2 + lax.axis_index("c")
    base = wid * ROWS_W
    pltpu.sync_copy(pos_hbm, pos_v)
    pltpu.sync_copy(w_hbm, w_v)
    pltpu.sync_copy(te_hbm, nt_v)
    n_total = nt_v[pl.ds(16, 16)][NT - 16] * B_TILE

    zi = jnp.zeros((16,), jnp.int32)
    zf = jnp.zeros((16,), jnp.float32)

    def _zero(i, _):
        tok_v[pl.ds(base + i * 16, 16)] = zi
        wr_v[pl.ds(base + i * 16, 16)] = zf
        return 0
    lax.fori_loop(0, ROWS_W // 16, _zero, 0, unroll=4)

    lane16 = lax.iota(jnp.int32, 16)

    def _scatter(i, _):
        idx = pos_v[pl.ds(i * 16, 16)]
        a = lane16 + i * 16
        tok = lax.bitwise_and(a, T - 1)
        plsc.store_scatter(tok_v, [idx], tok)
        plsc.store_scatter(wr_v, [idx], w_v[pl.ds(i * 16, 16)])
        return 0
    lax.fori_loop(0, A // 16, _scatter, 0, unroll=4)

    pltpu.sync_copy(wr_v.at[pl.ds(base, ROWS_W)],
                    wrow_hbm.at[pl.ds(base, ROWS_W)])

    active = n_total - base
    bufs = (buf0, buf1)
    sems = (sem0, sem1)
    nch = ROWS_W // GCH

    def _src(i):
        return x_hbm.at[tok_v.at[pl.ds(base + i * GCH, GCH)]]

    def _drain_out(j):
        @pl.when(j * GCH < active)
        def _():
            pltpu.make_async_copy(_src(j), bufs[j % 2], sems[j % 2]).wait()
            pltpu.sync_copy(bufs[j % 2],
                            xs_hbm.at[pl.ds(base + j * GCH, GCH)])

    for i in range(nch):
        @pl.when(i * GCH < active)
        def _(i=i):
            pltpu.async_copy(_src(i), bufs[i % 2], sems[i % 2])
        if i >= 1:
            _drain_out(i - 1)
    _drain_out(nch - 1)


def _dispatch_call(pos_flat, w_flat, flat, te_flat):
    mesh = plsc.VectorSubcoreMesh(core_axis_name="c", subcore_axis_name="s")
    f = pl.kernel(
        _dispatch_body,
        out_type=(jax.ShapeDtypeStruct((P, D), jnp.float32),
                  jax.ShapeDtypeStruct((P,), jnp.float32)),
        mesh=mesh,
        scratch_types=[
            pltpu.VMEM((A,), jnp.int32),
            pltpu.VMEM((A,), jnp.float32),
            pltpu.VMEM((P,), jnp.int32),
            pltpu.VMEM((P,), jnp.float32),
            pltpu.VMEM((NT_PAD,), jnp.int32),
            pltpu.VMEM((GCH, D), jnp.float32),
            pltpu.VMEM((GCH, D), jnp.float32),
            pltpu.SemaphoreType.DMA,
            pltpu.SemaphoreType.DMA,
        ],
        compiler_params=pltpu.CompilerParams(needs_layout_passes=False))
    return f(pos_flat, w_flat, flat, te_flat)


def _mlp_body(te_s, na_s, xs_ref, wg_ref, wi_ref, wo_ref, wr_ref,
              out_ref, acc_ref):
    c = pl.program_id(0)
    t = pl.program_id(1)

    @pl.when(t < na_s[0])
    def _():
        xb = xs_ref[...].astype(jnp.bfloat16)
        g = jnp.dot(xb, wg_ref[0].astype(jnp.bfloat16),
                    preferred_element_type=jnp.float32)
        u = jnp.dot(xb, wi_ref[0].astype(jnp.bfloat16),
                    preferred_element_type=jnp.float32)
        h = (g * jax.lax.logistic(g)) * u
        part = jnp.dot(h.astype(jnp.bfloat16), wo_ref[0].astype(jnp.bfloat16),
                       preferred_element_type=jnp.float32)
        sl = pl.ds(t * B_TILE, B_TILE)

        @pl.when(c == 0)
        def _a():
            acc_ref[sl, :] = part

        @pl.when(c > 0)
        def _b():
            acc_ref[sl, :] = acc_ref[sl, :] + part

        @pl.when(c == NC - 1)
        def _c():
            out_ref[...] = acc_ref[sl, :] * wr_ref[...]


def _mlp_call(te_arr, na_arr, xs, wg, wi, wo, w_row, interpret=False):
    grid_spec = pltpu.PrefetchScalarGridSpec(
        num_scalar_prefetch=2,
        grid=(NC, NT),
        in_specs=[
            pl.BlockSpec((B_TILE, D), lambda c, t, te, na: (t, 0)),
            pl.BlockSpec((1, D, FC), lambda c, t, te, na: (te[t], 0, c)),
            pl.BlockSpec((1, D, FC), lambda c, t, te, na: (te[t], 0, c)),
            pl.BlockSpec((1, FC, D), lambda c, t, te, na: (te[t], c, 0)),
            pl.BlockSpec((B_TILE, 1), lambda c, t, te, na: (t, 0)),
        ],
        out_specs=pl.BlockSpec((B_TILE, D), lambda c, t, te, na: (t, 0)),
        scratch_shapes=[pltpu.VMEM((P, D), jnp.float32)],
    )
    return pl.pallas_call(
        _mlp_body,
        grid_spec=grid_spec,
        out_shape=jax.ShapeDtypeStruct((P, D), jnp.float32),
        compiler_params=pltpu.CompilerParams(
            dimension_semantics=("arbitrary", "arbitrary"),
            vmem_limit_bytes=100 * 1024 * 1024),
        interpret=interpret,
    )(te_arr, na_arr, xs, wg, wi, wo, w_row)


def _combine_body(posf_hbm, ys_hbm, out_hbm,
                  p0_v, p1_v, r0, r1, sem0, sem1):
    wid = lax.axis_index("s") * 2 + lax.axis_index("c")
    base = wid * TOK_W

    def chunk(cix, _):
        off = base + cix * TCH
        pltpu.sync_copy(posf_hbm.at[pl.ds(off, TCH)], p0_v)
        pltpu.sync_copy(posf_hbm.at[pl.ds(T + off, TCH)], p1_v)
        cp0 = pltpu.async_copy(ys_hbm.at[p0_v], r0, sem0)
        cp1 = pltpu.async_copy(ys_hbm.at[p1_v], r1, sem1)
        cp0.wait()
        cp1.wait()

        def row(j, _):
            def vec(v, _):
                sl = pl.ds(v * 16, 16)
                r0[j, sl] = r0[j, sl] + r1[j, sl]
                return 0
            lax.fori_loop(0, D // 16, vec, 0, unroll=4)
            return 0
        lax.fori_loop(0, TCH, row, 0)
        pltpu.sync_copy(r0, out_hbm.at[pl.ds(off, TCH)])
        return 0
    lax.fori_loop(0, TOK_W // TCH, chunk, 0)


def _combine_call(pos_flat, ys):
    mesh = plsc.VectorSubcoreMesh(core_axis_name="c", subcore_axis_name="s")
    f = pl.kernel(
        _combine_body,
        out_type=jax.ShapeDtypeStruct((T, D), jnp.float32),
        mesh=mesh,
        scratch_types=[
            pltpu.VMEM((TCH,), jnp.int32),
            pltpu.VMEM((TCH,), jnp.int32),
            pltpu.VMEM((TCH, D), jnp.float32),
            pltpu.VMEM((TCH, D), jnp.float32),
            pltpu.SemaphoreType.DMA,
            pltpu.SemaphoreType.DMA,
        ],
        compiler_params=pltpu.CompilerParams(needs_layout_passes=False))
    return f(pos_flat, ys)


def kernel(x, W_router, W_gate, W_in, W_out):
    Bb, S, Dm = x.shape
    flat = x.reshape(T, D)
    logits = flat @ W_router
    posf, wf, te_full = _plan_call(logits)
    pos_flat = posf[:, 0]
    w_flat = wf[:, 0]
    xs, w_row = _dispatch_call(pos_flat, w_flat, flat, te_full[:, 0])
    te_arr = te_full[:NT, 0]
    na_arr = te_full[NT:NT + 1, 0]
    ys = _mlp_call(te_arr, na_arr, xs, W_gate, W_in, W_out,
                   w_row.reshape(P, 1))
    out = _combine_call(pos_flat, ys)
    return out.reshape(Bb, S, Dm)
